# baseline (speedup 1.0000x reference)
import functools

import jax
import jax.numpy as jnp
from jax import lax
from jax.experimental import pallas as pl
from jax.experimental.pallas import tpu as pltpu
from jax.experimental.pallas import tpu_sc as plsc

N = 10000
E = 320000
D = 128

NC = 2
NS = 16
LANES = 16

CHUNK = 128
HALF = 5120
N_PAD = 2 * HALF
RPH = HALF // NS
CAPC = 2560
CAPE = CAPC * CHUNK

_mesh = plsc.VectorSubcoreMesh(
    core_axis_name="c", subcore_axis_name="s", num_cores=NC, num_subcores=NS)


@functools.partial(
    pl.kernel,
    out_type=jax.ShapeDtypeStruct((N_PAD, D), jnp.float32),
    mesh=_mesh,
    scratch_types=[
        pltpu.VMEM((4, CHUNK, D), jnp.float32),
        pltpu.VMEM((4, 2, CHUNK), jnp.int32),
        pltpu.VMEM((4, LANES, CHUNK), jnp.float32),
        pltpu.VMEM((8, 128), jnp.int32),
        pltpu.VMEM_SHARED((HALF, D), jnp.float32),
        pltpu.SemaphoreType.DMA, pltpu.SemaphoreType.DMA,
        pltpu.SemaphoreType.DMA, pltpu.SemaphoreType.DMA,
        pltpu.SemaphoreType.DMA, pltpu.SemaphoreType.DMA,
        pltpu.SemaphoreType.DMA, pltpu.SemaphoreType.DMA,
        pltpu.SemaphoreType.DMA, pltpu.SemaphoreType.DMA,
        pltpu.SemaphoreType.DMA, pltpu.SemaphoreType.DMA,
    ],
)
def _spmm_sc(h_hbm, zeros_hbm, idx_hbm, w_hbm, cnt_hbm, out_hbm,
             rows_v, idx_v, w_v, cnt_v, acc_sh,
             sg0, sg1, sg2, sg3, si0, si1, si2, si3, ss0, ss1, ss2, ss3):
    c = lax.axis_index("c")
    s = lax.axis_index("s")
    sg = (sg0, sg1, sg2, sg3)
    si = (si0, si1, si2, si3)
    ss = (ss0, ss1, ss2, ss3)

    pltpu.sync_copy(cnt_hbm.at[c], cnt_v)
    cpw = cnt_v[0, pl.ds(0, LANES)][0]

    pltpu.sync_copy(zeros_hbm.at[pl.ds(s * RPH, RPH)],
                    acc_sh.at[pl.ds(s * RPH, RPH)])
    plsc.subcore_barrier()

    def meta_copy(slot, t):
        g = NS * t + s
        pltpu.async_copy(idx_hbm.at[c, g], idx_v.at[slot], si[slot])
        pltpu.async_copy(w_hbm.at[c, g], w_v.at[slot], si[slot])

    def meta_wait(slot, t):
        g = NS * t + s
        pltpu.make_async_copy(idx_hbm.at[c, g], idx_v.at[slot],
                              si[slot]).wait()
        pltpu.make_async_copy(w_hbm.at[c, g], w_v.at[slot],
                              si[slot]).wait()

    def gather_start(slot):
        pltpu.async_copy(h_hbm.at[idx_v.at[slot, 0]], rows_v.at[slot],
                         sg[slot])

    def gather_wait(slot):
        pltpu.make_async_copy(h_hbm.at[idx_v.at[slot, 0]], rows_v.at[slot],
                              sg[slot]).wait()

    def scatter_start(slot):
        pltpu.async_copy(rows_v.at[slot], acc_sh.at[idx_v.at[slot, 1]],
                         ss[slot], add=True)

    def scatter_wait(slot):
        pltpu.make_async_copy(rows_v.at[slot], acc_sh.at[idx_v.at[slot, 1]],
                              ss[slot]).wait()

    meta_copy(0, 0)
    meta_copy(1, 1)
    meta_wait(0, 0)
    gather_start(0)

    def step(j, b):
        nslot = (b + 1) % 4
        mslot = (b + 2) % 4
        @pl.when(j >= 2)
        def _():
            scatter_wait(mslot)

        @pl.when(j + 1 < cpw)
        def _():
            meta_wait(nslot, j + 1)
            gather_start(nslot)

        @pl.when(j + 2 < cpw)
        def _():
            meta_copy(mslot, j + 2)

        gather_wait(b)

        def edge_body(i, carry):
            wv = w_v[b, i // 8, pl.ds((i % 8) * LANES, LANES)]
            for jj in range(D // LANES):
                sl = (b, i, pl.ds(jj * LANES, LANES))
                rows_v[sl] = rows_v[sl] * wv
            return carry
        lax.fori_loop(0, CHUNK, edge_body, 0, unroll=8)

        scatter_start(b)

    def loop_body(jj, carry):
        for b in range(4):
            step(4 * jj + b, b)
        return carry
    lax.fori_loop(0, cpw // 4, loop_body, 0)

    scatter_wait(2)
    scatter_wait(3)
    plsc.subcore_barrier()

    pltpu.sync_copy(acc_sh.at[pl.ds(s * RPH, RPH)],
                    out_hbm.at[pl.ds(c * HALF + s * RPH, RPH)])


_BN = 1000


def _linear_tc(h, W, b2):
    def body(h_ref, w_ref, b_ref, o_ref):
        acc = lax.dot_general(h_ref[...], w_ref[...],
                              (((1,), (1,)), ((), ())),
                              preferred_element_type=jnp.float32)
        o_ref[...] = acc + b_ref[...]
    return pl.pallas_call(
        body,
        grid=(N // _BN,),
        in_specs=[
            pl.BlockSpec((_BN, D), lambda i: (i, 0)),
            pl.BlockSpec((D, D), lambda i: (0, 0)),
            pl.BlockSpec((1, D), lambda i: (0, 0)),
        ],
        out_specs=pl.BlockSpec((_BN, D), lambda i: (i, 0)),
        out_shape=jax.ShapeDtypeStruct((N, D), jnp.float32),
    )(h, W, b2)


def kernel(x, edge_index, edge_weight, W, b, k):
    row = edge_index[0]
    col = edge_index[1]

    side = row >= HALF
    sidx = jnp.cumsum(side.astype(jnp.int32))
    n1 = sidx[-1]
    n0 = E - n1
    ar = jnp.arange(E, dtype=jnp.int32)
    pos = jnp.where(side, sidx - 1, ar - sidx)
    tgt = jnp.where(side, CAPE + pos, pos)
    colp = jnp.zeros((2 * CAPE,), jnp.int32).at[tgt].set(col)
    rowp = jnp.zeros((2 * CAPE,), jnp.int32).at[tgt].set(
        row - side.astype(jnp.int32) * HALF)
    wp = jnp.zeros((2 * CAPE,), jnp.float32).at[tgt].set(edge_weight)

    idx = jnp.concatenate(
        [colp.reshape(2, CAPC, 1, CHUNK), rowp.reshape(2, CAPC, 1, CHUNK)],
        axis=2)
    wexp = jnp.broadcast_to(
        wp.reshape(2, CAPC, CHUNK, 1),
        (2, CAPC, CHUNK, LANES)).reshape(2, CAPC, LANES, CHUNK)

    def _cpw(n):
        chunks = (n + CHUNK - 1) // CHUNK
        return jnp.maximum(4 * ((chunks + 4 * NS - 1) // (4 * NS)), 4)
    cnts = jnp.broadcast_to(
        jnp.stack([_cpw(n0), _cpw(n1)]).astype(jnp.int32)[:, None, None],
        (2, 8, 128))

    zeros = jnp.zeros((HALF, D), jnp.float32)
    b2 = b.reshape(1, D)
    x_pad = jnp.pad(x, ((0, N_PAD - N), (0, 0)))

    def it_body(_, h):
        return _spmm_sc(h, zeros, idx, wexp, cnts)

    h = lax.fori_loop(0, k, it_body, x_pad)
    return _linear_tc(h[:N], W, b2)


# speedup vs baseline: 1.0056x; 1.0056x over previous
import functools

import jax
import jax.numpy as jnp
from jax import lax
from jax.experimental import pallas as pl
from jax.experimental.pallas import tpu as pltpu
from jax.experimental.pallas import tpu_sc as plsc

N = 10000
E = 320000
D = 128

NC = 2
NS = 16
LANES = 16

CHUNK = 128
HALF = 5120
N_PAD = 2 * HALF
RPH = HALF // NS
CAPC = 2560
CAPE = CAPC * CHUNK

_mesh = plsc.VectorSubcoreMesh(
    core_axis_name="c", subcore_axis_name="s", num_cores=NC, num_subcores=NS)


@functools.partial(
    pl.kernel,
    out_type=jax.ShapeDtypeStruct((N_PAD, D), jnp.float32),
    mesh=_mesh,
    scratch_types=[
        pltpu.VMEM((4, CHUNK, D), jnp.float32),
        pltpu.VMEM((4, 2, CHUNK), jnp.int32),
        pltpu.VMEM((4, LANES, CHUNK), jnp.float32),
        pltpu.VMEM((8, 128), jnp.int32),
        pltpu.VMEM_SHARED((HALF, D), jnp.float32),
        pltpu.SemaphoreType.DMA, pltpu.SemaphoreType.DMA,
        pltpu.SemaphoreType.DMA, pltpu.SemaphoreType.DMA,
        pltpu.SemaphoreType.DMA, pltpu.SemaphoreType.DMA,
        pltpu.SemaphoreType.DMA, pltpu.SemaphoreType.DMA,
        pltpu.SemaphoreType.DMA, pltpu.SemaphoreType.DMA,
        pltpu.SemaphoreType.DMA, pltpu.SemaphoreType.DMA,
    ],
)
def _spmm_sc(h_hbm, zeros_hbm, idx_hbm, w_hbm, cnt_hbm, out_hbm,
             rows_v, idx_v, w_v, cnt_v, acc_sh,
             sg0, sg1, sg2, sg3, si0, si1, si2, si3, ss0, ss1, ss2, ss3):
    c = lax.axis_index("c")
    s = lax.axis_index("s")
    sg = (sg0, sg1, sg2, sg3)
    si = (si0, si1, si2, si3)
    ss = (ss0, ss1, ss2, ss3)

    pltpu.sync_copy(cnt_hbm.at[c], cnt_v)
    cpw = cnt_v[0, pl.ds(0, LANES)][0]

    pltpu.sync_copy(zeros_hbm.at[pl.ds(s * RPH, RPH)],
                    acc_sh.at[pl.ds(s * RPH, RPH)])
    plsc.subcore_barrier()

    def meta_copy(slot, t):
        g = NS * t + s
        pltpu.async_copy(idx_hbm.at[c, g], idx_v.at[slot], si[slot])
        pltpu.async_copy(w_hbm.at[c, g], w_v.at[slot], si[slot])

    def meta_wait(slot, t):
        g = NS * t + s
        pltpu.make_async_copy(idx_hbm.at[c, g], idx_v.at[slot],
                              si[slot]).wait()
        pltpu.make_async_copy(w_hbm.at[c, g], w_v.at[slot],
                              si[slot]).wait()

    def gather_start(slot):
        pltpu.async_copy(h_hbm.at[idx_v.at[slot, 0]], rows_v.at[slot],
                         sg[slot])

    def gather_wait(slot):
        pltpu.make_async_copy(h_hbm.at[idx_v.at[slot, 0]], rows_v.at[slot],
                              sg[slot]).wait()

    def scatter_start(slot):
        pltpu.async_copy(rows_v.at[slot], acc_sh.at[idx_v.at[slot, 1]],
                         ss[slot], add=True)

    def scatter_wait(slot):
        pltpu.make_async_copy(rows_v.at[slot], acc_sh.at[idx_v.at[slot, 1]],
                              ss[slot]).wait()

    meta_copy(0, 0)
    meta_copy(1, 1)
    meta_wait(0, 0)
    gather_start(0)

    def step(j, b):
        nslot = (b + 1) % 4
        mslot = (b + 2) % 4
        @pl.when(j >= 2)
        def _():
            scatter_wait(mslot)

        @pl.when(j + 1 < cpw)
        def _():
            meta_wait(nslot, j + 1)
            gather_start(nslot)

        @pl.when(j + 2 < cpw)
        def _():
            meta_copy(mslot, j + 2)

        gather_wait(b)

        def edge_body(i, carry):
            wv = w_v[b, i // 8, pl.ds((i % 8) * LANES, LANES)]
            for jj in range(D // LANES):
                sl = (b, i, pl.ds(jj * LANES, LANES))
                rows_v[sl] = rows_v[sl] * wv
            return carry
        lax.fori_loop(0, CHUNK, edge_body, 0, unroll=2)

        scatter_start(b)

    def loop_body(jj, carry):
        for b in range(4):
            step(4 * jj + b, b)
        return carry
    lax.fori_loop(0, cpw // 4, loop_body, 0)

    scatter_wait(2)
    scatter_wait(3)
    plsc.subcore_barrier()

    pltpu.sync_copy(acc_sh.at[pl.ds(s * RPH, RPH)],
                    out_hbm.at[pl.ds(c * HALF + s * RPH, RPH)])


_BN = 1000


def _linear_tc(h, W, b2):
    def body(h_ref, w_ref, b_ref, o_ref):
        acc = lax.dot_general(h_ref[...], w_ref[...],
                              (((1,), (1,)), ((), ())),
                              preferred_element_type=jnp.float32)
        o_ref[...] = acc + b_ref[...]
    return pl.pallas_call(
        body,
        grid=(N // _BN,),
        in_specs=[
            pl.BlockSpec((_BN, D), lambda i: (i, 0)),
            pl.BlockSpec((D, D), lambda i: (0, 0)),
            pl.BlockSpec((1, D), lambda i: (0, 0)),
        ],
        out_specs=pl.BlockSpec((_BN, D), lambda i: (i, 0)),
        out_shape=jax.ShapeDtypeStruct((N, D), jnp.float32),
    )(h, W, b2)


def kernel(x, edge_index, edge_weight, W, b, k):
    row = edge_index[0]
    col = edge_index[1]

    side = row >= HALF
    sidx = jnp.cumsum(side.astype(jnp.int32))
    n1 = sidx[-1]
    n0 = E - n1
    ar = jnp.arange(E, dtype=jnp.int32)
    pos = jnp.where(side, sidx - 1, ar - sidx)
    tgt = jnp.where(side, CAPE + pos, pos)
    colp = jnp.zeros((2 * CAPE,), jnp.int32).at[tgt].set(col)
    rowp = jnp.zeros((2 * CAPE,), jnp.int32).at[tgt].set(
        row - side.astype(jnp.int32) * HALF)
    wp = jnp.zeros((2 * CAPE,), jnp.float32).at[tgt].set(edge_weight)

    idx = jnp.concatenate(
        [colp.reshape(2, CAPC, 1, CHUNK), rowp.reshape(2, CAPC, 1, CHUNK)],
        axis=2)
    wexp = jnp.broadcast_to(
        wp.reshape(2, CAPC, CHUNK, 1),
        (2, CAPC, CHUNK, LANES)).reshape(2, CAPC, LANES, CHUNK)

    def _cpw(n):
        chunks = (n + CHUNK - 1) // CHUNK
        return jnp.maximum(4 * ((chunks + 4 * NS - 1) // (4 * NS)), 4)
    cnts = jnp.broadcast_to(
        jnp.stack([_cpw(n0), _cpw(n1)]).astype(jnp.int32)[:, None, None],
        (2, 8, 128))

    zeros = jnp.zeros((HALF, D), jnp.float32)
    b2 = b.reshape(1, D)
    x_pad = jnp.pad(x, ((0, N_PAD - N), (0, 0)))

    def it_body(_, h):
        return _spmm_sc(h, zeros, idx, wexp, cnts)

    h = lax.fori_loop(0, k, it_body, x_pad)
    return _linear_tc(h[:N], W, b2)
